# baseline (device time: 113127 ns/iter reference)
import jax
import jax.numpy as jnp
from jax import lax
from jax.experimental import pallas as pl
from jax.experimental.pallas import tpu as pltpu

N_DEV = 8
N_HOP = N_DEV - 1
B, Sq, Hq, Dh = 2, 256, 8, 64
Dmodel = 768
BH = B * Hq
SCALE = 0.125


def kernel(x, Wq, Wo, K_ext, V_ext):
    Skv = K_ext.shape[1]
    K2 = K_ext.reshape(B, Skv, Hq * Dh)
    V2 = V_ext.reshape(B, Skv, Hq * Dh)

    def body(x_ref, wq_ref, wo_ref, k_ref, v_ref, out_ref,
             commo_ref, commml_ref, acc_o_ref, acc_m_ref, acc_l_ref,
             osend_sems, orecv_sems, mlsend_sems, mlrecv_sems):
        my = lax.axis_index("i")
        left = lax.rem(my + (N_DEV - 1), N_DEV)
        right = lax.rem(my + 1, N_DEV)

        barrier = pltpu.get_barrier_semaphore()
        pl.semaphore_signal(barrier, inc=1, device_id=(left,),
                            device_id_type=pl.DeviceIdType.MESH)
        pl.semaphore_signal(barrier, inc=1, device_id=(right,),
                            device_id_type=pl.DeviceIdType.MESH)
        pl.semaphore_wait(barrier, 2)

        wq = wq_ref[...].astype(jnp.bfloat16)
        for b in range(B):
            xb = x_ref[b].astype(jnp.bfloat16)
            q = lax.dot_general(xb, wq, (((1,), (0,)), ((), ())),
                                preferred_element_type=jnp.float32)
            kb = k_ref[b].astype(jnp.bfloat16)
            vb = v_ref[b].astype(jnp.bfloat16)
            for h in range(Hq):
                idx = b * Hq + h
                qh = (q[:, h * Dh:(h + 1) * Dh] * SCALE).astype(jnp.bfloat16)
                kh = kb[:, h * Dh:(h + 1) * Dh]
                vh = vb[:, h * Dh:(h + 1) * Dh]
                sT = lax.dot_general(kh, qh, (((1,), (1,)), ((), ())),
                                     preferred_element_type=jnp.float32)
                m = jnp.max(sT, axis=0, keepdims=True)
                p = jnp.exp(sT - m)
                lsum = jnp.sum(p, axis=0, keepdims=True)
                oT = lax.dot_general(vh, p.astype(jnp.bfloat16),
                                     (((0,), (0,)), ((), ())),
                                     preferred_element_type=jnp.float32)
                acc_o_ref[idx] = oT
                commo_ref[0, idx] = oT
                acc_m_ref[idx:idx + 1, :] = m
                acc_l_ref[idx:idx + 1, :] = lsum
                commml_ref[0, 0, idx:idx + 1, :] = m
                commml_ref[0, 1, idx:idx + 1, :] = lsum

        for h in range(N_HOP):
            s_slot = h % 2
            r_slot = (h + 1) % 2
            rdma_o = pltpu.make_async_remote_copy(
                src_ref=commo_ref.at[s_slot],
                dst_ref=commo_ref.at[r_slot],
                send_sem=osend_sems.at[s_slot],
                recv_sem=orecv_sems.at[h],
                device_id=(right,),
                device_id_type=pl.DeviceIdType.MESH,
            )
            rdma_ml = pltpu.make_async_remote_copy(
                src_ref=commml_ref.at[s_slot],
                dst_ref=commml_ref.at[r_slot],
                send_sem=mlsend_sems.at[s_slot],
                recv_sem=mlrecv_sems.at[h],
                device_id=(right,),
                device_id_type=pl.DeviceIdType.MESH,
            )
            rdma_o.start()
            rdma_ml.start()
            rdma_o.wait()
            rdma_ml.wait()

            in_m = commml_ref[r_slot, 0]
            in_l = commml_ref[r_slot, 1]
            cur_m = acc_m_ref[...]
            cur_l = acc_l_ref[...]
            m_new = jnp.maximum(cur_m, in_m)
            fa = jnp.exp(cur_m - m_new)
            fb = jnp.exp(in_m - m_new)
            acc_m_ref[...] = m_new
            acc_l_ref[...] = cur_l * fa + in_l * fb
            acc_o_ref[...] = (acc_o_ref[...] * fa[:, None, :]
                              + commo_ref[r_slot] * fb[:, None, :])

        inv_l = 1.0 / acc_l_ref[...]
        for b in range(B):
            acc = jnp.zeros((Sq, Dmodel), jnp.float32)
            for h in range(Hq):
                idx = b * Hq + h
                oT_n = (acc_o_ref[idx] * inv_l[idx:idx + 1, :]).astype(jnp.bfloat16)
                woh = wo_ref[h * Dh:(h + 1) * Dh, :].astype(jnp.bfloat16)
                acc = acc + lax.dot_general(oT_n, woh, (((0,), (0,)), ((), ())),
                                            preferred_element_type=jnp.float32)
            out_ref[b] = acc

    return pl.pallas_call(
        body,
        out_shape=jax.ShapeDtypeStruct((B, Sq, Dmodel), jnp.float32),
        in_specs=[pl.BlockSpec(memory_space=pltpu.VMEM)] * 5,
        out_specs=pl.BlockSpec(memory_space=pltpu.VMEM),
        scratch_shapes=[
            pltpu.VMEM((2, BH, Dh, Sq), jnp.float32),
            pltpu.VMEM((2, 2, BH, Sq), jnp.float32),
            pltpu.VMEM((BH, Dh, Sq), jnp.float32),
            pltpu.VMEM((BH, Sq), jnp.float32),
            pltpu.VMEM((BH, Sq), jnp.float32),
            pltpu.SemaphoreType.DMA((2,)),
            pltpu.SemaphoreType.DMA((N_HOP,)),
            pltpu.SemaphoreType.DMA((2,)),
            pltpu.SemaphoreType.DMA((N_HOP,)),
        ],
        compiler_params=pltpu.CompilerParams(collective_id=0),
    )(x, Wq, Wo, K2, V2)


# device time: 74148 ns/iter; 1.5257x vs baseline; 1.5257x over previous
import jax
import jax.numpy as jnp
from jax import lax
from jax.experimental import pallas as pl
from jax.experimental.pallas import tpu as pltpu

N_DEV = 8
N_HOP = N_DEV - 1
B, Sq, Hq, Dh = 2, 256, 8, 64
Dmodel = 768
BH = B * Hq
SCALE = 0.125


def kernel(x, Wq, Wo, K_ext, V_ext):
    Skv = K_ext.shape[1]
    K2 = K_ext.reshape(B, Skv, Hq * Dh)
    V2 = V_ext.reshape(B, Skv, Hq * Dh)

    def body(x_ref, wq_ref, wo_ref, k_ref, v_ref, out_ref,
             commo_ref, commml_ref, acc_o_ref, acc_m_ref, acc_l_ref,
             osend_sems, orecv_sems, mlsend_sems, mlrecv_sems):
        my = lax.axis_index("i")
        left = lax.rem(my + (N_DEV - 1), N_DEV)
        right = lax.rem(my + 1, N_DEV)

        barrier = pltpu.get_barrier_semaphore()
        pl.semaphore_signal(barrier, inc=1, device_id=(left,),
                            device_id_type=pl.DeviceIdType.MESH)
        pl.semaphore_signal(barrier, inc=1, device_id=(right,),
                            device_id_type=pl.DeviceIdType.MESH)
        pl.semaphore_wait(barrier, 2)

        wq = wq_ref[...].astype(jnp.bfloat16)
        for b in range(B):
            xb = x_ref[b].astype(jnp.bfloat16)
            q = lax.dot_general(xb, wq, (((1,), (0,)), ((), ())),
                                preferred_element_type=jnp.float32)
            kb = k_ref[b].astype(jnp.bfloat16)
            vb = v_ref[b].astype(jnp.bfloat16)
            for h in range(Hq):
                idx = b * Hq + h
                qh = (q[:, h * Dh:(h + 1) * Dh] * SCALE).astype(jnp.bfloat16)
                kh = kb[:, h * Dh:(h + 1) * Dh]
                vh = vb[:, h * Dh:(h + 1) * Dh]
                sT = lax.dot_general(kh, qh, (((1,), (1,)), ((), ())),
                                     preferred_element_type=jnp.float32)
                m = jnp.max(sT, axis=0, keepdims=True)
                p = jnp.exp(sT - m)
                lsum = jnp.sum(p, axis=0, keepdims=True)
                oT = lax.dot_general(vh, p.astype(jnp.bfloat16),
                                     (((0,), (0,)), ((), ())),
                                     preferred_element_type=jnp.float32)
                acc_o_ref[idx] = oT
                commo_ref[0, idx] = oT.astype(jnp.bfloat16)
                acc_m_ref[idx:idx + 1, :] = m
                acc_l_ref[idx:idx + 1, :] = lsum
                commml_ref[0, 0, idx:idx + 1, :] = m
                commml_ref[0, 1, idx:idx + 1, :] = lsum

        for h in range(N_HOP):
            s_slot = h % 2
            r_slot = (h + 1) % 2
            rdma_o = pltpu.make_async_remote_copy(
                src_ref=commo_ref.at[s_slot],
                dst_ref=commo_ref.at[r_slot],
                send_sem=osend_sems.at[s_slot],
                recv_sem=orecv_sems.at[h],
                device_id=(right,),
                device_id_type=pl.DeviceIdType.MESH,
            )
            rdma_ml = pltpu.make_async_remote_copy(
                src_ref=commml_ref.at[s_slot],
                dst_ref=commml_ref.at[r_slot],
                send_sem=mlsend_sems.at[s_slot],
                recv_sem=mlrecv_sems.at[h],
                device_id=(right,),
                device_id_type=pl.DeviceIdType.MESH,
            )
            rdma_o.start()
            rdma_ml.start()
            rdma_o.wait()
            rdma_ml.wait()

            in_m = commml_ref[r_slot, 0]
            in_l = commml_ref[r_slot, 1]
            cur_m = acc_m_ref[...]
            cur_l = acc_l_ref[...]
            m_new = jnp.maximum(cur_m, in_m)
            fa = jnp.exp(cur_m - m_new)
            fb = jnp.exp(in_m - m_new)
            acc_m_ref[...] = m_new
            acc_l_ref[...] = cur_l * fa + in_l * fb
            acc_o_ref[...] = (acc_o_ref[...] * fa[:, None, :]
                              + commo_ref[r_slot].astype(jnp.float32)
                              * fb[:, None, :])

        inv_l = 1.0 / acc_l_ref[...]
        for b in range(B):
            acc = jnp.zeros((Sq, Dmodel), jnp.float32)
            for h in range(Hq):
                idx = b * Hq + h
                oT_n = (acc_o_ref[idx] * inv_l[idx:idx + 1, :]).astype(jnp.bfloat16)
                woh = wo_ref[h * Dh:(h + 1) * Dh, :].astype(jnp.bfloat16)
                acc = acc + lax.dot_general(oT_n, woh, (((0,), (0,)), ((), ())),
                                            preferred_element_type=jnp.float32)
            out_ref[b] = acc

    return pl.pallas_call(
        body,
        out_shape=jax.ShapeDtypeStruct((B, Sq, Dmodel), jnp.float32),
        in_specs=[pl.BlockSpec(memory_space=pltpu.VMEM)] * 5,
        out_specs=pl.BlockSpec(memory_space=pltpu.VMEM),
        scratch_shapes=[
            pltpu.VMEM((2, BH, Dh, Sq), jnp.bfloat16),
            pltpu.VMEM((2, 2, BH, Sq), jnp.float32),
            pltpu.VMEM((BH, Dh, Sq), jnp.float32),
            pltpu.VMEM((BH, Sq), jnp.float32),
            pltpu.VMEM((BH, Sq), jnp.float32),
            pltpu.SemaphoreType.DMA((2,)),
            pltpu.SemaphoreType.DMA((N_HOP,)),
            pltpu.SemaphoreType.DMA((2,)),
            pltpu.SemaphoreType.DMA((N_HOP,)),
        ],
        compiler_params=pltpu.CompilerParams(collective_id=0),
    )(x, Wq, Wo, K2, V2)


# device time: 42643 ns/iter; 2.6529x vs baseline; 1.7388x over previous
import jax
import jax.numpy as jnp
from jax import lax
from jax.experimental import pallas as pl
from jax.experimental.pallas import tpu as pltpu

N_DEV = 8
DISTS = (1, 2, 4)
B, Sq, Hq, Dh = 2, 256, 8, 64
Dmodel = 768
BH = B * Hq
SCALE = 0.125


def kernel(x, Wq, Wo, K_ext, V_ext):
    Skv = K_ext.shape[1]
    K2 = K_ext.reshape(B, Skv, Hq * Dh)
    V2 = V_ext.reshape(B, Skv, Hq * Dh)

    def body(x_ref, wq_ref, wo_ref, k_ref, v_ref, out_ref,
             sendo_ref, sendml_ref, recvo_ref, recvml_ref,
             acc_o_ref, acc_m_ref, acc_l_ref,
             osend_sems, orecv_sems, mlsend_sems, mlrecv_sems):
        my = lax.axis_index("i")

        barrier = pltpu.get_barrier_semaphore()
        for dist in DISTS:
            pl.semaphore_signal(barrier, inc=1, device_id=(my ^ dist,),
                                device_id_type=pl.DeviceIdType.MESH)
        pl.semaphore_wait(barrier, len(DISTS))

        wq = wq_ref[...].astype(jnp.bfloat16)
        for b in range(B):
            xb = x_ref[b].astype(jnp.bfloat16)
            q = lax.dot_general(xb, wq, (((1,), (0,)), ((), ())),
                                preferred_element_type=jnp.float32)
            kb = k_ref[b].astype(jnp.bfloat16)
            vb = v_ref[b].astype(jnp.bfloat16)
            for h in range(Hq):
                idx = b * Hq + h
                qh = (q[:, h * Dh:(h + 1) * Dh] * SCALE).astype(jnp.bfloat16)
                kh = kb[:, h * Dh:(h + 1) * Dh]
                vh = vb[:, h * Dh:(h + 1) * Dh]
                sT = lax.dot_general(kh, qh, (((1,), (1,)), ((), ())),
                                     preferred_element_type=jnp.float32)
                m = jnp.max(sT, axis=0, keepdims=True)
                p = jnp.exp(sT - m)
                lsum = jnp.sum(p, axis=0, keepdims=True)
                oT = lax.dot_general(vh, p.astype(jnp.bfloat16),
                                     (((0,), (0,)), ((), ())),
                                     preferred_element_type=jnp.float32)
                acc_o_ref[idx] = oT
                sendo_ref[idx] = oT.astype(jnp.bfloat16)
                acc_m_ref[idx:idx + 1, :] = m
                acc_l_ref[idx:idx + 1, :] = lsum
                sendml_ref[0, idx:idx + 1, :] = m
                sendml_ref[1, idx:idx + 1, :] = lsum

        for r, dist in enumerate(DISTS):
            partner = my ^ dist
            rdma_o = pltpu.make_async_remote_copy(
                src_ref=sendo_ref,
                dst_ref=recvo_ref.at[r],
                send_sem=osend_sems.at[r],
                recv_sem=orecv_sems.at[r],
                device_id=(partner,),
                device_id_type=pl.DeviceIdType.MESH,
            )
            rdma_ml = pltpu.make_async_remote_copy(
                src_ref=sendml_ref,
                dst_ref=recvml_ref.at[r],
                send_sem=mlsend_sems.at[r],
                recv_sem=mlrecv_sems.at[r],
                device_id=(partner,),
                device_id_type=pl.DeviceIdType.MESH,
            )
            rdma_o.start()
            rdma_ml.start()
            rdma_o.wait()
            rdma_ml.wait()

            in_m = recvml_ref[r, 0]
            in_l = recvml_ref[r, 1]
            cur_m = acc_m_ref[...]
            cur_l = acc_l_ref[...]
            m_new = jnp.maximum(cur_m, in_m)
            fa = jnp.exp(cur_m - m_new)
            fb = jnp.exp(in_m - m_new)
            l_new = cur_l * fa + in_l * fb
            o_new = (acc_o_ref[...] * fa[:, None, :]
                     + recvo_ref[r].astype(jnp.float32) * fb[:, None, :])
            acc_m_ref[...] = m_new
            acc_l_ref[...] = l_new
            acc_o_ref[...] = o_new
            if r < len(DISTS) - 1:
                sendo_ref[...] = o_new.astype(jnp.bfloat16)
                sendml_ref[0] = m_new
                sendml_ref[1] = l_new

        inv_l = 1.0 / acc_l_ref[...]
        for b in range(B):
            acc = jnp.zeros((Sq, Dmodel), jnp.float32)
            for h in range(Hq):
                idx = b * Hq + h
                oT_n = (acc_o_ref[idx] * inv_l[idx:idx + 1, :]).astype(jnp.bfloat16)
                woh = wo_ref[h * Dh:(h + 1) * Dh, :].astype(jnp.bfloat16)
                acc = acc + lax.dot_general(oT_n, woh, (((0,), (0,)), ((), ())),
                                            preferred_element_type=jnp.float32)
            out_ref[b] = acc

    n_r = len(DISTS)
    return pl.pallas_call(
        body,
        out_shape=jax.ShapeDtypeStruct((B, Sq, Dmodel), jnp.float32),
        in_specs=[pl.BlockSpec(memory_space=pltpu.VMEM)] * 5,
        out_specs=pl.BlockSpec(memory_space=pltpu.VMEM),
        scratch_shapes=[
            pltpu.VMEM((BH, Dh, Sq), jnp.bfloat16),
            pltpu.VMEM((2, BH, Sq), jnp.float32),
            pltpu.VMEM((n_r, BH, Dh, Sq), jnp.bfloat16),
            pltpu.VMEM((n_r, 2, BH, Sq), jnp.float32),
            pltpu.VMEM((BH, Dh, Sq), jnp.float32),
            pltpu.VMEM((BH, Sq), jnp.float32),
            pltpu.VMEM((BH, Sq), jnp.float32),
            pltpu.SemaphoreType.DMA((n_r,)),
            pltpu.SemaphoreType.DMA((n_r,)),
            pltpu.SemaphoreType.DMA((n_r,)),
            pltpu.SemaphoreType.DMA((n_r,)),
        ],
        compiler_params=pltpu.CompilerParams(collective_id=0),
    )(x, Wq, Wo, K2, V2)


# device time: 40086 ns/iter; 2.8221x vs baseline; 1.0638x over previous
import jax
import jax.numpy as jnp
from jax import lax
from jax.experimental import pallas as pl
from jax.experimental.pallas import tpu as pltpu

N_DEV = 8
RS_DISTS = (4, 2, 1)
RS_SIZES = (8, 4, 2)
AG_DISTS = (1, 2, 4)
AG_SIZES = (2, 4, 8)
B, Sq, Hq, Dh = 2, 256, 8, 64
Dmodel = 768
BH = B * Hq
SCALE = 0.125


def kernel(x, Wq, Wo, K_ext, V_ext):
    Skv = K_ext.shape[1]
    K2 = K_ext.reshape(B, Skv, Hq * Dh)
    V2 = V_ext.reshape(B, Skv, Hq * Dh)

    def body(x_ref, wq_ref, wo_ref, k_ref, v_ref, out_ref,
             sendo_ref, sendml_ref,
             recvo0_ref, recvo1_ref, recvo2_ref,
             recvml0_ref, recvml1_ref, recvml2_ref,
             go_ref, acc_o_ref, acc_ml_ref,
             rs_osend, rs_orecv, rs_mlsend, rs_mlrecv, ag_send, ag_recv):
        my = lax.axis_index("i")
        bit2 = my // 4
        bit1 = (my // 2) % 2
        bit0 = my % 2

        barrier = pltpu.get_barrier_semaphore()
        for dist in (1, 2, 4):
            pl.semaphore_signal(barrier, inc=1, device_id=(my ^ dist,),
                                device_id_type=pl.DeviceIdType.MESH)
        pl.semaphore_wait(barrier, 3)

        wq = wq_ref[...].astype(jnp.bfloat16)
        for b in range(B):
            xb = x_ref[b].astype(jnp.bfloat16)
            q = lax.dot_general(xb, wq, (((1,), (0,)), ((), ())),
                                preferred_element_type=jnp.float32)
            kb = k_ref[b].astype(jnp.bfloat16)
            vb = v_ref[b].astype(jnp.bfloat16)
            for h in range(Hq):
                idx = b * Hq + h
                qh = (q[:, h * Dh:(h + 1) * Dh] * SCALE).astype(jnp.bfloat16)
                kh = kb[:, h * Dh:(h + 1) * Dh]
                vh = vb[:, h * Dh:(h + 1) * Dh]
                sT = lax.dot_general(kh, qh, (((1,), (1,)), ((), ())),
                                     preferred_element_type=jnp.float32)
                m = jnp.max(sT, axis=0, keepdims=True)
                p = jnp.exp(sT - m)
                lsum = jnp.sum(p, axis=0, keepdims=True)
                oT = lax.dot_general(vh, p.astype(jnp.bfloat16),
                                     (((0,), (0,)), ((), ())),
                                     preferred_element_type=jnp.float32)
                acc_o_ref[idx] = oT
                acc_ml_ref[idx, 0:1, :] = m
                acc_ml_ref[idx, 1:2, :] = lsum

        recvo_refs = (recvo0_ref, recvo1_ref, recvo2_ref)
        recvml_refs = (recvml0_ref, recvml1_ref, recvml2_ref)
        bits = (bit2, bit1, bit0)
        klo = 0
        for r, (dist, size) in enumerate(zip(RS_DISTS, RS_SIZES)):
            partner = my ^ dist
            slo = klo + (1 - bits[r]) * size
            klo = klo + bits[r] * size

            sendo_ref[pl.ds(0, size)] = (
                acc_o_ref[pl.ds(slo, size)].astype(jnp.bfloat16))
            sendml_ref[pl.ds(0, size)] = acc_ml_ref[pl.ds(slo, size)]

            rdma_o = pltpu.make_async_remote_copy(
                src_ref=sendo_ref.at[pl.ds(0, size)],
                dst_ref=recvo_refs[r],
                send_sem=rs_osend.at[r],
                recv_sem=rs_orecv.at[r],
                device_id=(partner,),
                device_id_type=pl.DeviceIdType.MESH,
            )
            rdma_ml = pltpu.make_async_remote_copy(
                src_ref=sendml_ref.at[pl.ds(0, size)],
                dst_ref=recvml_refs[r],
                send_sem=rs_mlsend.at[r],
                recv_sem=rs_mlrecv.at[r],
                device_id=(partner,),
                device_id_type=pl.DeviceIdType.MESH,
            )
            rdma_o.start()
            rdma_ml.start()
            rdma_o.wait()
            rdma_ml.wait()

            in_ml = recvml_refs[r][...]
            in_m = in_ml[:, 0, :]
            in_l = in_ml[:, 1, :]
            cur_ml = acc_ml_ref[pl.ds(klo, size)]
            cur_m = cur_ml[:, 0, :]
            cur_l = cur_ml[:, 1, :]
            m_new = jnp.maximum(cur_m, in_m)
            fa = jnp.exp(cur_m - m_new)
            fb = jnp.exp(in_m - m_new)
            l_new = cur_l * fa + in_l * fb
            acc_ml_ref[pl.ds(klo, size)] = jnp.concatenate(
                [m_new[:, None, :], l_new[:, None, :]], axis=1)
            acc_o_ref[pl.ds(klo, size)] = (
                acc_o_ref[pl.ds(klo, size)] * fa[:, None, :]
                + recvo_refs[r][...].astype(jnp.float32) * fb[:, None, :])

        plo = my * 2
        l_piece = acc_ml_ref[pl.ds(plo, 2)][:, 1, :]
        o_piece = acc_o_ref[pl.ds(plo, 2)]
        go_ref[pl.ds(plo, 2)] = (
            o_piece / l_piece[:, None, :]).astype(jnp.bfloat16)

        glo = plo
        for r, (dist, size) in enumerate(zip(AG_DISTS, AG_SIZES)):
            rdma = pltpu.make_async_remote_copy(
                src_ref=go_ref.at[pl.ds(glo, size)],
                dst_ref=go_ref.at[pl.ds(glo, size)],
                send_sem=ag_send.at[r],
                recv_sem=ag_recv.at[r],
                device_id=(my ^ dist,),
                device_id_type=pl.DeviceIdType.MESH,
            )
            rdma.start()
            rdma.wait()
            glo = glo - bits[2 - r] * size

        for b in range(B):
            acc = jnp.zeros((Sq, Dmodel), jnp.float32)
            for h in range(Hq):
                idx = b * Hq + h
                oT_n = go_ref[idx]
                woh = wo_ref[h * Dh:(h + 1) * Dh, :].astype(jnp.bfloat16)
                acc = acc + lax.dot_general(oT_n, woh, (((0,), (0,)), ((), ())),
                                            preferred_element_type=jnp.float32)
            out_ref[b] = acc

    return pl.pallas_call(
        body,
        out_shape=jax.ShapeDtypeStruct((B, Sq, Dmodel), jnp.float32),
        in_specs=[pl.BlockSpec(memory_space=pltpu.VMEM)] * 5,
        out_specs=pl.BlockSpec(memory_space=pltpu.VMEM),
        scratch_shapes=[
            pltpu.VMEM((8, Dh, Sq), jnp.bfloat16),
            pltpu.VMEM((8, 2, Sq), jnp.float32),
            pltpu.VMEM((8, Dh, Sq), jnp.bfloat16),
            pltpu.VMEM((4, Dh, Sq), jnp.bfloat16),
            pltpu.VMEM((2, Dh, Sq), jnp.bfloat16),
            pltpu.VMEM((8, 2, Sq), jnp.float32),
            pltpu.VMEM((4, 2, Sq), jnp.float32),
            pltpu.VMEM((2, 2, Sq), jnp.float32),
            pltpu.VMEM((BH, Dh, Sq), jnp.bfloat16),
            pltpu.VMEM((BH, Dh, Sq), jnp.float32),
            pltpu.VMEM((BH, 2, Sq), jnp.float32),
            pltpu.SemaphoreType.DMA((3,)),
            pltpu.SemaphoreType.DMA((3,)),
            pltpu.SemaphoreType.DMA((3,)),
            pltpu.SemaphoreType.DMA((3,)),
            pltpu.SemaphoreType.DMA((3,)),
            pltpu.SemaphoreType.DMA((3,)),
        ],
        compiler_params=pltpu.CompilerParams(collective_id=0),
    )(x, Wq, Wo, K2, V2)


# device time: 31116 ns/iter; 3.6357x vs baseline; 1.2883x over previous
import jax
import jax.numpy as jnp
from jax import lax
from jax.experimental import pallas as pl
from jax.experimental.pallas import tpu as pltpu

N_DEV = 8
B, Sq, Hq, Dh = 2, 256, 8, 64
Dmodel = 768
BH = B * Hq
ROWS = BH // N_DEV
SCALE = 0.125


def kernel(x, Wq, Wo, K_ext, V_ext):
    Skv = K_ext.shape[1]
    K2 = K_ext.reshape(B, Skv, Hq * Dh)
    V2 = V_ext.reshape(B, Skv, Hq * Dh)

    def body(x_ref, wq_ref, wo_ref, k_ref, v_ref, out_ref,
             sendo_ref, go_ref, recvo_ref, recvml_ref,
             acc_o_ref, acc_ml_ref,
             rs_osend, rs_orecv, rs_mlsend, rs_mlrecv, ag_send, ag_recv):
        my = lax.axis_index("i")

        barrier = pltpu.get_barrier_semaphore()
        for k in range(1, N_DEV):
            pl.semaphore_signal(barrier, inc=1, device_id=(my ^ k,),
                                device_id_type=pl.DeviceIdType.MESH)
        pl.semaphore_wait(barrier, N_DEV - 1)

        wq = wq_ref[...].astype(jnp.bfloat16)
        for b in range(B):
            xb = x_ref[b].astype(jnp.bfloat16)
            q = lax.dot_general(xb, wq, (((1,), (0,)), ((), ())),
                                preferred_element_type=jnp.float32)
            kb = k_ref[b].astype(jnp.bfloat16)
            vb = v_ref[b].astype(jnp.bfloat16)
            for h in range(Hq):
                idx = b * Hq + h
                qh = (q[:, h * Dh:(h + 1) * Dh] * SCALE).astype(jnp.bfloat16)
                kh = kb[:, h * Dh:(h + 1) * Dh]
                vh = vb[:, h * Dh:(h + 1) * Dh]
                sT = lax.dot_general(kh, qh, (((1,), (1,)), ((), ())),
                                     preferred_element_type=jnp.float32)
                m = jnp.max(sT, axis=0, keepdims=True)
                p = jnp.exp(sT - m)
                lsum = jnp.sum(p, axis=0, keepdims=True)
                oT = lax.dot_general(vh, p.astype(jnp.bfloat16),
                                     (((0,), (0,)), ((), ())),
                                     preferred_element_type=jnp.float32)
                acc_o_ref[idx] = oT
                sendo_ref[idx] = oT.astype(jnp.bfloat16)
                acc_ml_ref[idx, 0:1, :] = m
                acc_ml_ref[idx, 1:2, :] = lsum

        rs_o = []
        rs_ml = []
        for k in range(1, N_DEV):
            partner = my ^ k
            prow = partner * ROWS
            ro = pltpu.make_async_remote_copy(
                src_ref=sendo_ref.at[pl.ds(prow, ROWS)],
                dst_ref=recvo_ref.at[k],
                send_sem=rs_osend.at[k],
                recv_sem=rs_orecv.at[k],
                device_id=(partner,),
                device_id_type=pl.DeviceIdType.MESH,
            )
            rml = pltpu.make_async_remote_copy(
                src_ref=acc_ml_ref.at[pl.ds(prow, ROWS)],
                dst_ref=recvml_ref.at[k],
                send_sem=rs_mlsend.at[k],
                recv_sem=rs_mlrecv.at[k],
                device_id=(partner,),
                device_id_type=pl.DeviceIdType.MESH,
            )
            ro.start()
            rml.start()
            rs_o.append(ro)
            rs_ml.append(rml)

        plo = my * ROWS
        cur_ml = acc_ml_ref[pl.ds(plo, ROWS)]
        cm = cur_ml[:, 0, :]
        cl = cur_ml[:, 1, :]
        co = acc_o_ref[pl.ds(plo, ROWS)]
        for k in range(1, N_DEV):
            rs_o[k - 1].wait_recv()
            rs_ml[k - 1].wait_recv()
            in_ml = recvml_ref[k]
            im = in_ml[:, 0, :]
            il = in_ml[:, 1, :]
            mn = jnp.maximum(cm, im)
            fa = jnp.exp(cm - mn)
            fb = jnp.exp(im - mn)
            cl = cl * fa + il * fb
            co = (co * fa[:, None, :]
                  + recvo_ref[k].astype(jnp.float32) * fb[:, None, :])
            cm = mn

        go_ref[pl.ds(plo, ROWS)] = (co / cl[:, None, :]).astype(jnp.bfloat16)

        ag = []
        for k in range(1, N_DEV):
            r = pltpu.make_async_remote_copy(
                src_ref=go_ref.at[pl.ds(plo, ROWS)],
                dst_ref=go_ref.at[pl.ds(plo, ROWS)],
                send_sem=ag_send.at[k],
                recv_sem=ag_recv.at[k],
                device_id=(my ^ k,),
                device_id_type=pl.DeviceIdType.MESH,
            )
            r.start()
            ag.append(r)

        for r in rs_o + rs_ml:
            r.wait_send()
        for r in ag:
            r.wait_recv()
        for r in ag:
            r.wait_send()

        for b in range(B):
            acc = jnp.zeros((Sq, Dmodel), jnp.float32)
            for h in range(Hq):
                idx = b * Hq + h
                oT_n = go_ref[idx]
                woh = wo_ref[h * Dh:(h + 1) * Dh, :].astype(jnp.bfloat16)
                acc = acc + lax.dot_general(oT_n, woh, (((0,), (0,)), ((), ())),
                                            preferred_element_type=jnp.float32)
            out_ref[b] = acc

    return pl.pallas_call(
        body,
        out_shape=jax.ShapeDtypeStruct((B, Sq, Dmodel), jnp.float32),
        in_specs=[pl.BlockSpec(memory_space=pltpu.VMEM)] * 5,
        out_specs=pl.BlockSpec(memory_space=pltpu.VMEM),
        scratch_shapes=[
            pltpu.VMEM((BH, Dh, Sq), jnp.bfloat16),
            pltpu.VMEM((BH, Dh, Sq), jnp.bfloat16),
            pltpu.VMEM((N_DEV, ROWS, Dh, Sq), jnp.bfloat16),
            pltpu.VMEM((N_DEV, ROWS, 2, Sq), jnp.float32),
            pltpu.VMEM((BH, Dh, Sq), jnp.float32),
            pltpu.VMEM((BH, 2, Sq), jnp.float32),
            pltpu.SemaphoreType.DMA((N_DEV,)),
            pltpu.SemaphoreType.DMA((N_DEV,)),
            pltpu.SemaphoreType.DMA((N_DEV,)),
            pltpu.SemaphoreType.DMA((N_DEV,)),
            pltpu.SemaphoreType.DMA((N_DEV,)),
            pltpu.SemaphoreType.DMA((N_DEV,)),
        ],
        compiler_params=pltpu.CompilerParams(collective_id=0),
    )(x, Wq, Wo, K2, V2)
